# baseline (speedup 1.0000x reference)
import functools

import jax
import jax.numpy as jnp
from jax import lax
from jax.experimental import pallas as pl
from jax.experimental.pallas import tpu as pltpu
from jax.experimental.pallas import tpu_sc as plsc

B = 1024
D = 256
L = 100000
K = 4096
NEG = 32
CLOSE = 16

NC = 2
NS = 16
NW = NC * NS
BPW = B // NW
NEG_PER_W = B * NEG // NW
NEG_CHUNK = 128
N_NEG_CHUNKS = NEG_PER_W // NEG_CHUNK

BLK = 256
NB = B // BLK


def _sc_gather_body(idx_hbm, negidx_hbm, bank_hbm, labank_hbm, cent_hbm,
                    labels_out, poscent_out, inspos_out, insneg_out,
                    idx_v, labels_v, poscent_v, inspos_v,
                    negidx_v, negbuf_v, sem_a, sem_b):
    wid = lax.axis_index("s") * NC + lax.axis_index("c")
    base = wid * BPW

    pltpu.sync_copy(idx_hbm.at[wid], idx_v)

    pltpu.async_copy(labank_hbm.at[idx_v], labels_v, sem_a).wait()
    pltpu.sync_copy(labels_v, labels_out.at[pl.ds(base, BPW)])
    pltpu.async_copy(cent_hbm.at[labels_v], poscent_v, sem_a).wait()
    pltpu.sync_copy(poscent_v, poscent_out.at[pl.ds(base, BPW)])

    pltpu.async_copy(bank_hbm.at[idx_v], inspos_v, sem_a).wait()
    pltpu.sync_copy(inspos_v, inspos_out.at[pl.ds(base, BPW)])

    nbase = wid * NEG_PER_W
    pltpu.sync_copy(negidx_hbm.at[wid], negidx_v)
    sems = (sem_a, sem_b)
    copies = [pltpu.async_copy(bank_hbm.at[negidx_v.at[0]],
                               negbuf_v.at[0], sem_a)]
    for c in range(N_NEG_CHUNKS):
        if c + 1 < N_NEG_CHUNKS:
            copies.append(
                pltpu.async_copy(bank_hbm.at[negidx_v.at[c + 1]],
                                 negbuf_v.at[(c + 1) % 2],
                                 sems[(c + 1) % 2]))
        copies[c].wait()
        pltpu.sync_copy(
            negbuf_v.at[c % 2],
            insneg_out.at[pl.ds(nbase + c * NEG_CHUNK, NEG_CHUNK)])


@functools.cache
def _make_sc_gather():
    return pl.kernel(
        _sc_gather_body,
        out_type=[
            jax.ShapeDtypeStruct((B,), jnp.int32),
            jax.ShapeDtypeStruct((B, D), jnp.float32),
            jax.ShapeDtypeStruct((B, D), jnp.float32),
            jax.ShapeDtypeStruct((B * NEG, D), jnp.float32),
        ],
        mesh=plsc.VectorSubcoreMesh(core_axis_name="c", subcore_axis_name="s"),
        scratch_types=[
            pltpu.VMEM((BPW,), jnp.int32),
            pltpu.VMEM((BPW,), jnp.int32),
            pltpu.VMEM((BPW, D), jnp.float32),
            pltpu.VMEM((BPW, D), jnp.float32),
            pltpu.VMEM((N_NEG_CHUNKS, NEG_CHUNK), jnp.int32),
            pltpu.VMEM((2, NEG_CHUNK, D), jnp.float32),
            pltpu.SemaphoreType.DMA,
            pltpu.SemaphoreType.DMA,
        ],
    )


def _tc_body(feat_ref, poscent_ref, inspos_ref, insneg_ref, labs_ref,
             cent_ref, ips_ref, ins_ref, cps_ref, cnsT_ref):
    f = feat_ref[...]
    pc = poscent_ref[...]

    ips_ref[...] = jnp.sum(f * inspos_ref[...], axis=1, keepdims=True)
    cps_ref[...] = jnp.sum(f * pc, axis=1, keepdims=True)
    ins_ref[...] = jnp.sum(insneg_ref[...] * f[:, None, :], axis=2)

    cent = cent_ref[...]
    dims = (((1,), (1,)), ((), ()))
    sqcol = jnp.sum(cent * cent, axis=1, keepdims=True)
    cpcT = lax.dot_general(cent, pc, dims,
                           preferred_element_type=jnp.float32)
    simsT = lax.dot_general(cent, f, dims,
                            preferred_element_type=jnp.float32)

    labs = labs_ref[0]
    kio = lax.broadcasted_iota(jnp.int32, (K, BLK), 0)
    big = jnp.float32(3.0e38)
    selfmask = kio == labs
    sq_pos = jnp.min(jnp.where(selfmask, sqcol, big), axis=0,
                     keepdims=True)
    d2 = (sq_pos + sqcol) - 2.0 * cpcT
    dist = jnp.sqrt(jnp.maximum(d2, 0.0))
    dist = jnp.where(selfmask, big, dist)

    key = lax.bitcast_convert_type(dist, jnp.int32)
    ibig = jnp.int32(0x7FFFFFFF)
    rows = []
    m = jnp.min(key, axis=0, keepdims=True)
    idxv = jnp.min(jnp.where(key == m, kio, K), axis=0, keepdims=True)
    for j in range(CLOSE):
        eqi = (key == m) & (kio == idxv)
        rows.append(jnp.sum(jnp.where(eqi, simsT, 0.0), axis=0,
                            keepdims=True))
        if j + 1 < CLOSE:
            after = (key > m) | ((key == m) & (kio > idxv))
            m = jnp.min(jnp.where(after, key, ibig), axis=0, keepdims=True)
            idxv = jnp.min(jnp.where((key == m) & after, kio, K), axis=0,
                           keepdims=True)
    cnsT_ref[...] = jnp.concatenate(rows, axis=0)


def _make_tc(interpret=False):
    return pl.pallas_call(
        _tc_body,
        grid=(NB,),
        in_specs=[
            pl.BlockSpec((BLK, D), lambda i: (i, 0)),
            pl.BlockSpec((BLK, D), lambda i: (i, 0)),
            pl.BlockSpec((BLK, D), lambda i: (i, 0)),
            pl.BlockSpec((BLK, NEG, D), lambda i: (i, 0, 0)),
            pl.BlockSpec((1, 1, BLK), lambda i: (i, 0, 0)),
            pl.BlockSpec((K, D), lambda i: (0, 0)),
        ],
        out_specs=[
            pl.BlockSpec((BLK, 1), lambda i: (i, 0)),
            pl.BlockSpec((BLK, NEG), lambda i: (i, 0)),
            pl.BlockSpec((BLK, 1), lambda i: (i, 0)),
            pl.BlockSpec((CLOSE, BLK), lambda i: (0, i)),
        ],
        out_shape=[
            jax.ShapeDtypeStruct((B, 1), jnp.float32),
            jax.ShapeDtypeStruct((B, NEG), jnp.float32),
            jax.ShapeDtypeStruct((B, 1), jnp.float32),
            jax.ShapeDtypeStruct((CLOSE, B), jnp.float32),
        ],
        interpret=interpret,
    )


_tc_call = _make_tc()


@jax.jit
def kernel(feature, idx, neg_indices, feature_bank, label_bank, centroids):
    idx = idx.astype(jnp.int32)
    labels, poscent, inspos, insneg = _make_sc_gather()(
        idx.reshape(NW, BPW),
        neg_indices.reshape(NW, N_NEG_CHUNKS, NEG_CHUNK),
        feature_bank, label_bank, centroids)
    ips, ins, cps, cnsT = _tc_call(feature, poscent, inspos,
                                   insneg.reshape(B, NEG, D),
                                   labels.reshape(NB, 1, BLK), centroids)
    return ips, ins, cps, cnsT.T


# speedup vs baseline: 1.0795x; 1.0795x over previous
import functools

import jax
import jax.numpy as jnp
from jax import lax
from jax.experimental import pallas as pl
from jax.experimental.pallas import tpu as pltpu
from jax.experimental.pallas import tpu_sc as plsc

B = 1024
D = 256
L = 100000
K = 4096
NEG = 32
CLOSE = 16

NC = 2
NS = 16
NW = NC * NS
BPW = B // NW
NEG_PER_W = B * NEG // NW
NEG_CHUNK = 128
N_NEG_CHUNKS = NEG_PER_W // NEG_CHUNK

BLK = 128
NB = B // BLK


def _sc_gather_body(idx_hbm, negidx_hbm, bank_hbm, labank_hbm, cent_hbm,
                    labels_out, poscent_out, inspos_out, insneg_out,
                    idx_v, labels_v, poscent_v, inspos_v,
                    negidx_v, negbuf_v, sem_a, sem_b):
    wid = lax.axis_index("s") * NC + lax.axis_index("c")
    base = wid * BPW

    pltpu.sync_copy(idx_hbm.at[wid], idx_v)

    pltpu.async_copy(labank_hbm.at[idx_v], labels_v, sem_a).wait()
    pltpu.sync_copy(labels_v, labels_out.at[pl.ds(base, BPW)])
    pltpu.async_copy(cent_hbm.at[labels_v], poscent_v, sem_a).wait()
    pltpu.sync_copy(poscent_v, poscent_out.at[pl.ds(base, BPW)])

    pltpu.async_copy(bank_hbm.at[idx_v], inspos_v, sem_a).wait()
    pltpu.sync_copy(inspos_v, inspos_out.at[pl.ds(base, BPW)])

    nbase = wid * NEG_PER_W
    pltpu.sync_copy(negidx_hbm.at[wid], negidx_v)
    sems = (sem_a, sem_b)
    copies = [pltpu.async_copy(bank_hbm.at[negidx_v.at[0]],
                               negbuf_v.at[0], sem_a)]
    for c in range(N_NEG_CHUNKS):
        if c + 1 < N_NEG_CHUNKS:
            copies.append(
                pltpu.async_copy(bank_hbm.at[negidx_v.at[c + 1]],
                                 negbuf_v.at[(c + 1) % 2],
                                 sems[(c + 1) % 2]))
        copies[c].wait()
        pltpu.sync_copy(
            negbuf_v.at[c % 2],
            insneg_out.at[pl.ds(nbase + c * NEG_CHUNK, NEG_CHUNK)])


@functools.cache
def _make_sc_gather():
    return pl.kernel(
        _sc_gather_body,
        out_type=[
            jax.ShapeDtypeStruct((B,), jnp.int32),
            jax.ShapeDtypeStruct((B, D), jnp.float32),
            jax.ShapeDtypeStruct((B, D), jnp.float32),
            jax.ShapeDtypeStruct((B * NEG, D), jnp.float32),
        ],
        mesh=plsc.VectorSubcoreMesh(core_axis_name="c", subcore_axis_name="s"),
        scratch_types=[
            pltpu.VMEM((BPW,), jnp.int32),
            pltpu.VMEM((BPW,), jnp.int32),
            pltpu.VMEM((BPW, D), jnp.float32),
            pltpu.VMEM((BPW, D), jnp.float32),
            pltpu.VMEM((N_NEG_CHUNKS, NEG_CHUNK), jnp.int32),
            pltpu.VMEM((2, NEG_CHUNK, D), jnp.float32),
            pltpu.SemaphoreType.DMA,
            pltpu.SemaphoreType.DMA,
        ],
    )


def _tc_body(feat_ref, poscent_ref, inspos_ref, insneg_ref, labs_ref,
             cent_ref, ips_ref, ins_ref, cps_ref, cnsT_ref):
    f = feat_ref[...]
    pc = poscent_ref[...]

    ips_ref[...] = jnp.sum(f * inspos_ref[...], axis=1, keepdims=True)
    cps_ref[...] = jnp.sum(f * pc, axis=1, keepdims=True)
    ins_ref[...] = jnp.sum(insneg_ref[...] * f[:, None, :], axis=2)

    cent = cent_ref[...]
    dims = (((1,), (1,)), ((), ()))
    sqcol = jnp.sum(cent * cent, axis=1, keepdims=True)
    cpcT = lax.dot_general(cent, pc, dims,
                           preferred_element_type=jnp.float32)
    simsT = lax.dot_general(cent, f, dims,
                            preferred_element_type=jnp.float32)

    labs = labs_ref[0]
    kio = lax.broadcasted_iota(jnp.int32, (K, BLK), 0)
    big = jnp.float32(3.0e38)
    selfmask = kio == labs
    sq_pos = jnp.min(jnp.where(selfmask, sqcol, big), axis=0,
                     keepdims=True)
    d2 = (sq_pos + sqcol) - 2.0 * cpcT
    dist = jnp.sqrt(jnp.maximum(d2, 0.0))
    dist = jnp.where(selfmask, big, dist)

    key = lax.bitcast_convert_type(dist, jnp.int32)
    ibig = jnp.int32(0x7FFFFFFF)
    rows = []
    m = jnp.min(key, axis=0, keepdims=True)
    idxv = jnp.min(jnp.where(key == m, kio, K), axis=0, keepdims=True)
    for j in range(CLOSE):
        eqi = (key == m) & (kio == idxv)
        rows.append(jnp.sum(jnp.where(eqi, simsT, 0.0), axis=0,
                            keepdims=True))
        if j + 1 < CLOSE:
            after = (key > m) | ((key == m) & (kio > idxv))
            m = jnp.min(jnp.where(after, key, ibig), axis=0, keepdims=True)
            idxv = jnp.min(jnp.where((key == m) & after, kio, K), axis=0,
                           keepdims=True)
    cnsT_ref[...] = jnp.concatenate(rows, axis=0)


def _make_tc(interpret=False):
    return pl.pallas_call(
        _tc_body,
        grid=(NB,),
        in_specs=[
            pl.BlockSpec((BLK, D), lambda i: (i, 0)),
            pl.BlockSpec((BLK, D), lambda i: (i, 0)),
            pl.BlockSpec((BLK, D), lambda i: (i, 0)),
            pl.BlockSpec((BLK, NEG, D), lambda i: (i, 0, 0)),
            pl.BlockSpec((1, 1, BLK), lambda i: (i, 0, 0)),
            pl.BlockSpec((K, D), lambda i: (0, 0)),
        ],
        out_specs=[
            pl.BlockSpec((BLK, 1), lambda i: (i, 0)),
            pl.BlockSpec((BLK, NEG), lambda i: (i, 0)),
            pl.BlockSpec((BLK, 1), lambda i: (i, 0)),
            pl.BlockSpec((CLOSE, BLK), lambda i: (0, i)),
        ],
        out_shape=[
            jax.ShapeDtypeStruct((B, 1), jnp.float32),
            jax.ShapeDtypeStruct((B, NEG), jnp.float32),
            jax.ShapeDtypeStruct((B, 1), jnp.float32),
            jax.ShapeDtypeStruct((CLOSE, B), jnp.float32),
        ],
        interpret=interpret,
    )


_tc_call = _make_tc()


@jax.jit
def kernel(feature, idx, neg_indices, feature_bank, label_bank, centroids):
    idx = idx.astype(jnp.int32)
    labels, poscent, inspos, insneg = _make_sc_gather()(
        idx.reshape(NW, BPW),
        neg_indices.reshape(NW, N_NEG_CHUNKS, NEG_CHUNK),
        feature_bank, label_bank, centroids)
    ips, ins, cps, cnsT = _tc_call(feature, poscent, inspos,
                                   insneg.reshape(B, NEG, D),
                                   labels.reshape(NB, 1, BLK), centroids)
    return ips, ins, cps, cnsT.T


# speedup vs baseline: 1.4192x; 1.3147x over previous
import functools

import jax
import jax.numpy as jnp
from jax import lax
from jax.experimental import pallas as pl
from jax.experimental.pallas import tpu as pltpu
from jax.experimental.pallas import tpu_sc as plsc

B = 1024
D = 256
L = 100000
K = 4096
NEG = 32
CLOSE = 16

NC = 2
NS = 16
NW = NC * NS
BPW = B // NW
NEG_PER_W = B * NEG // NW
NEG_CHUNK = 128
N_NEG_CHUNKS = NEG_PER_W // NEG_CHUNK

BLK = 128
NB = B // BLK


def _sc_gather_body(idx_hbm, negidx_hbm, bank_hbm, labank_hbm, cent_hbm,
                    labels_out, poscent_out, inspos_out, insneg_out,
                    idx_v, labels_v, poscent_v, inspos_v,
                    negidx_v, negbuf_v, sem_a, sem_b):
    wid = lax.axis_index("s") * NC + lax.axis_index("c")
    base = wid * BPW

    pltpu.sync_copy(idx_hbm.at[wid], idx_v)

    pltpu.async_copy(labank_hbm.at[idx_v], labels_v, sem_a).wait()
    pltpu.sync_copy(labels_v, labels_out.at[pl.ds(base, BPW)])
    pltpu.async_copy(cent_hbm.at[labels_v], poscent_v, sem_a).wait()
    pltpu.sync_copy(poscent_v, poscent_out.at[pl.ds(base, BPW)])

    pltpu.async_copy(bank_hbm.at[idx_v], inspos_v, sem_a).wait()
    pltpu.sync_copy(inspos_v, inspos_out.at[pl.ds(base, BPW)])

    nbase = wid * NEG_PER_W
    pltpu.sync_copy(negidx_hbm.at[wid], negidx_v)
    sems = (sem_a, sem_b)
    copies = [pltpu.async_copy(bank_hbm.at[negidx_v.at[0]],
                               negbuf_v.at[0], sem_a)]
    for c in range(N_NEG_CHUNKS):
        if c + 1 < N_NEG_CHUNKS:
            copies.append(
                pltpu.async_copy(bank_hbm.at[negidx_v.at[c + 1]],
                                 negbuf_v.at[(c + 1) % 2],
                                 sems[(c + 1) % 2]))
        copies[c].wait()
        pltpu.sync_copy(
            negbuf_v.at[c % 2],
            insneg_out.at[pl.ds(nbase + c * NEG_CHUNK, NEG_CHUNK)])


@functools.cache
def _make_sc_gather():
    return pl.kernel(
        _sc_gather_body,
        out_type=[
            jax.ShapeDtypeStruct((B,), jnp.int32),
            jax.ShapeDtypeStruct((B, D), jnp.float32),
            jax.ShapeDtypeStruct((B, D), jnp.float32),
            jax.ShapeDtypeStruct((B * NEG, D), jnp.float32),
        ],
        mesh=plsc.VectorSubcoreMesh(core_axis_name="c", subcore_axis_name="s"),
        scratch_types=[
            pltpu.VMEM((BPW,), jnp.int32),
            pltpu.VMEM((BPW,), jnp.int32),
            pltpu.VMEM((BPW, D), jnp.float32),
            pltpu.VMEM((BPW, D), jnp.float32),
            pltpu.VMEM((N_NEG_CHUNKS, NEG_CHUNK), jnp.int32),
            pltpu.VMEM((2, NEG_CHUNK, D), jnp.float32),
            pltpu.SemaphoreType.DMA,
            pltpu.SemaphoreType.DMA,
        ],
    )


def _tc_body(feat_ref, poscent_ref, inspos_ref, insneg_ref, labs_ref,
             cent_ref, ips_ref, ins_ref, cps_ref, cnsT_ref):
    f = feat_ref[...]
    pc = poscent_ref[...]

    ips_ref[...] = jnp.sum(f * inspos_ref[...], axis=1, keepdims=True)
    cps_ref[...] = jnp.sum(f * pc, axis=1, keepdims=True)
    ins_ref[...] = jnp.sum(insneg_ref[...] * f[:, None, :], axis=2)

    cent = cent_ref[...]
    dims = (((1,), (1,)), ((), ()))
    sqcol = jnp.sum(cent * cent, axis=1, keepdims=True)
    cpcT = lax.dot_general(cent, pc, dims,
                           preferred_element_type=jnp.float32)
    simsT = lax.dot_general(cent, f, dims,
                            preferred_element_type=jnp.float32)

    labs = labs_ref[0]
    kio = lax.broadcasted_iota(jnp.int32, (K, BLK), 0)
    big = jnp.float32(3.0e38)
    selfmask = kio == labs
    sq_pos = jnp.min(jnp.where(selfmask, sqcol, big), axis=0,
                     keepdims=True)
    d2 = (sq_pos + sqcol) - 2.0 * cpcT
    dist = jnp.sqrt(jnp.maximum(d2, 0.0))
    dist = jnp.where(selfmask, big, dist)

    key = lax.bitcast_convert_type(dist, jnp.int32)
    ibig = jnp.int32(0x7FFFFFFF)
    for j in range(CLOSE):
        m = jnp.min(key, axis=0, keepdims=True)
        hit = key == m
        idxv = jnp.min(jnp.where(hit, kio, K), axis=0, keepdims=True)
        eqi = hit & (kio == idxv)
        cnsT_ref[j:j + 1, :] = jnp.sum(jnp.where(eqi, simsT, 0.0), axis=0,
                                       keepdims=True)
        key = jnp.where(eqi, ibig, key)


def _make_tc(interpret=False):
    return pl.pallas_call(
        _tc_body,
        grid=(NB,),
        in_specs=[
            pl.BlockSpec((BLK, D), lambda i: (i, 0)),
            pl.BlockSpec((BLK, D), lambda i: (i, 0)),
            pl.BlockSpec((BLK, D), lambda i: (i, 0)),
            pl.BlockSpec((BLK, NEG, D), lambda i: (i, 0, 0)),
            pl.BlockSpec((1, 1, BLK), lambda i: (i, 0, 0)),
            pl.BlockSpec((K, D), lambda i: (0, 0)),
        ],
        out_specs=[
            pl.BlockSpec((BLK, 1), lambda i: (i, 0)),
            pl.BlockSpec((BLK, NEG), lambda i: (i, 0)),
            pl.BlockSpec((BLK, 1), lambda i: (i, 0)),
            pl.BlockSpec((CLOSE, BLK), lambda i: (0, i)),
        ],
        out_shape=[
            jax.ShapeDtypeStruct((B, 1), jnp.float32),
            jax.ShapeDtypeStruct((B, NEG), jnp.float32),
            jax.ShapeDtypeStruct((B, 1), jnp.float32),
            jax.ShapeDtypeStruct((CLOSE, B), jnp.float32),
        ],
        interpret=interpret,
    )


_tc_call = _make_tc()


@jax.jit
def kernel(feature, idx, neg_indices, feature_bank, label_bank, centroids):
    idx = idx.astype(jnp.int32)
    labels, poscent, inspos, insneg = _make_sc_gather()(
        idx.reshape(NW, BPW),
        neg_indices.reshape(NW, N_NEG_CHUNKS, NEG_CHUNK),
        feature_bank, label_bank, centroids)
    ips, ins, cps, cnsT = _tc_call(feature, poscent, inspos,
                                   insneg.reshape(B, NEG, D),
                                   labels.reshape(NB, 1, BLK), centroids)
    return ips, ins, cps, cnsT.T
